# Initial kernel scaffold; baseline (speedup 1.0000x reference)
import functools

import jax
import jax.numpy as jnp
from jax import lax
from jax.experimental import pallas as pl
from jax.experimental.pallas import tpu as pltpu
from jax.experimental.pallas import tpu_sc as plsc

N = 10000
E = 320000
D_IN = 128
HID = 128
NCLS = 16

NC = 2
NS = 16
NW = NC * NS
EPW = E // NW
CH = 80
NCHUNK = EPW // CH
SLAB = N // NS
NPAD = 10240
DSLAB = NPAD // NS

_mesh = plsc.VectorSubcoreMesh(core_axis_name="c", subcore_axis_name="s")


@functools.partial(
    pl.kernel,
    out_type=jax.ShapeDtypeStruct((NC, NPAD), jnp.float32),
    mesh=_mesh,
    scratch_types=[
        pltpu.VMEM((NCHUNK, CH), jnp.int32),
        pltpu.VMEM((CH,), jnp.float32),
        pltpu.VMEM((NPAD,), jnp.float32),
        pltpu.SemaphoreType.DMA,
    ],
)
def _deg_kernel(dst_hbm, out_hbm, dst_v, ones_v, z_v, sem):
    cid = lax.axis_index("c")
    sid = lax.axis_index("s")
    wid = sid * NC + cid
    pltpu.async_copy(dst_hbm.at[wid], dst_v, sem).wait()
    for i in range(CH // 16):
        ones_v[pl.ds(i * 16, 16)] = jnp.full((16,), 1.0, jnp.float32)

    @pl.when(sid == 0)
    def _():
        def zrow(i, c):
            z_v[pl.ds(i * 16, 16)] = jnp.zeros((16,), jnp.float32)
            return c
        lax.fori_loop(0, NPAD // 16, zrow, 0)

    def run(acc_sh):
        @pl.when(sid == 0)
        def _():
            pltpu.sync_copy(z_v, acc_sh)
        plsc.subcore_barrier()

        def body(j, c):
            pltpu.sync_copy(ones_v, acc_sh.at[dst_v.at[j]], add=True)
            return c
        lax.fori_loop(0, NCHUNK, body, 0)
        plsc.subcore_barrier()
        pltpu.sync_copy(acc_sh.at[pl.ds(sid * DSLAB, DSLAB)],
                        out_hbm.at[cid, pl.ds(sid * DSLAB, DSLAB)])

    pl.run_scoped(run, pltpu.VMEM_SHARED((NPAD,), jnp.float32))


def _make_agg_kernel(d):
    @functools.partial(
        pl.kernel,
        out_type=jax.ShapeDtypeStruct((NC, N, d), jnp.float32),
        mesh=_mesh,
        scratch_types=[
            pltpu.VMEM((NCHUNK, CH), jnp.int32),
            pltpu.VMEM((NCHUNK, CH), jnp.int32),
            pltpu.VMEM((CH, d), jnp.float32),
            pltpu.SemaphoreType.DMA,
        ],
    )
    def agg(h_hbm, src_hbm, dst_hbm, out_hbm, src_v, dst_v, rows_v, sem):
        cid = lax.axis_index("c")
        sid = lax.axis_index("s")
        wid = sid * NC + cid
        pltpu.async_copy(src_hbm.at[wid], src_v, sem).wait()
        pltpu.async_copy(dst_hbm.at[wid], dst_v, sem).wait()

        def run(acc_sh):
            pltpu.sync_copy(h_hbm.at[pl.ds(sid * SLAB, SLAB)],
                            acc_sh.at[pl.ds(sid * SLAB, SLAB)])
            plsc.subcore_barrier()

            def body(j, c):
                pltpu.async_copy(h_hbm.at[src_v.at[j]], rows_v, sem).wait()
                pltpu.sync_copy(rows_v, acc_sh.at[dst_v.at[j]], add=True)
                return c
            lax.fori_loop(0, NCHUNK, body, 0)
            plsc.subcore_barrier()
            pltpu.sync_copy(acc_sh.at[pl.ds(sid * SLAB, SLAB)],
                            out_hbm.at[cid, pl.ds(sid * SLAB, SLAB)])

        pl.run_scoped(run, pltpu.VMEM_SHARED((N, d), jnp.float32))

    return agg


_agg128 = _make_agg_kernel(HID)
_agg16 = _make_agg_kernel(NCLS)


_R = 1000


def _tc1_body(deg2_ref, x_ref, w1_ref, h1s_ref, dinv_ref):
    deg = deg2_ref[0] + deg2_ref[1] + 1.0
    dinv = lax.rsqrt(deg)
    dinv_ref[...] = dinv
    h = jnp.dot(x_ref[...], w1_ref[...], preferred_element_type=jnp.float32)
    h1s_ref[...] = h * dinv


def _tc1(deg2, x, W1):
    return pl.pallas_call(
        _tc1_body,
        grid=(N // _R,),
        in_specs=[
            pl.BlockSpec((NC, _R, 1), lambda i: (0, i, 0)),
            pl.BlockSpec((_R, D_IN), lambda i: (i, 0)),
            pl.BlockSpec((D_IN, HID), lambda i: (0, 0)),
        ],
        out_specs=[
            pl.BlockSpec((_R, HID), lambda i: (i, 0)),
            pl.BlockSpec((_R, 1), lambda i: (i, 0)),
        ],
        out_shape=[
            jax.ShapeDtypeStruct((N, HID), jnp.float32),
            jax.ShapeDtypeStruct((N, 1), jnp.float32),
        ],
    )(deg2, x, W1)


def _tc2_body(p_ref, h1s_ref, dinv_ref, b1_ref, w2_ref, h2s_ref):
    agg = p_ref[0] + p_ref[1] - h1s_ref[...]
    h1 = jnp.maximum(agg * dinv_ref[...] + b1_ref[...], 0.0)
    h2 = jnp.dot(h1, w2_ref[...], preferred_element_type=jnp.float32)
    h2s_ref[...] = h2 * dinv_ref[...]


def _tc2(p1, h1s, dinv, b1, W2):
    return pl.pallas_call(
        _tc2_body,
        grid=(N // _R,),
        in_specs=[
            pl.BlockSpec((NC, _R, HID), lambda i: (0, i, 0)),
            pl.BlockSpec((_R, HID), lambda i: (i, 0)),
            pl.BlockSpec((_R, 1), lambda i: (i, 0)),
            pl.BlockSpec((1, HID), lambda i: (0, 0)),
            pl.BlockSpec((HID, NCLS), lambda i: (0, 0)),
        ],
        out_specs=pl.BlockSpec((_R, NCLS), lambda i: (i, 0)),
        out_shape=jax.ShapeDtypeStruct((N, NCLS), jnp.float32),
    )(p1, h1s, dinv, b1, W2)


def _tc3_body(p_ref, h2s_ref, dinv_ref, b2_ref, out_ref):
    agg = p_ref[0] + p_ref[1] - h2s_ref[...]
    out_ref[...] = agg * dinv_ref[...] + b2_ref[...]


def _tc3(p2, h2s, dinv, b2):
    return pl.pallas_call(
        _tc3_body,
        grid=(N // _R,),
        in_specs=[
            pl.BlockSpec((NC, _R, NCLS), lambda i: (0, i, 0)),
            pl.BlockSpec((_R, NCLS), lambda i: (i, 0)),
            pl.BlockSpec((_R, 1), lambda i: (i, 0)),
            pl.BlockSpec((1, NCLS), lambda i: (0, 0)),
        ],
        out_specs=pl.BlockSpec((_R, NCLS), lambda i: (i, 0)),
        out_shape=jax.ShapeDtypeStruct((N, NCLS), jnp.float32),
    )(p2, h2s, dinv, b2)


def kernel(x, edge_index, W1, b1, W2, b2):
    src = edge_index[0].astype(jnp.int32).reshape(NW, NCHUNK, CH)
    dst = edge_index[1].astype(jnp.int32).reshape(NW, NCHUNK, CH)

    deg2 = _deg_kernel(dst)
    deg2 = deg2.reshape(NC, NPAD, 1)

    h1s, dinv = _tc1(deg2, x, W1)
    p1 = _agg128(h1s, src, dst)
    h2s = _tc2(p1, h1s, dinv, b1.reshape(1, HID), W2)
    p2 = _agg16(h2s, src, dst)
    out = _tc3(p2, h2s, dinv, b2.reshape(1, NCLS))
    return out


# speedup vs baseline: 22.9969x; 22.9969x over previous
import functools

import jax
import jax.numpy as jnp
from jax import lax
from jax.experimental import pallas as pl
from jax.experimental.pallas import tpu as pltpu
from jax.experimental.pallas import tpu_sc as plsc

N = 10000
E = 320000
D_IN = 128
HID = 128
NCLS = 16

NC = 2
NS = 16
NW = NC * NS
EPW = E // NW
CH = 80
NCHUNK = EPW // CH
SLAB = 624
TAIL = N - NS * SLAB
TAIL_OFF = NS * SLAB
NPAD = 10240
DSLAB = NPAD // NS

_mesh = plsc.VectorSubcoreMesh(core_axis_name="c", subcore_axis_name="s")


@functools.partial(
    pl.kernel,
    out_type=jax.ShapeDtypeStruct((NC, NPAD), jnp.float32),
    mesh=_mesh,
    compiler_params=pltpu.CompilerParams(use_tc_tiling_on_sc=False),
    scratch_types=[pltpu.VMEM_SHARED((NPAD,), jnp.float32)],
)
def _deg_kernel(dst_hbm, out_hbm, acc_sh):
    cid = lax.axis_index("c")
    sid = lax.axis_index("s")
    wid = sid * NC + cid

    def run(dst_v, ones_v, z_v, sem):
        pltpu.async_copy(dst_hbm.at[wid], dst_v, sem).wait()
        for i in range(CH // 16):
            ones_v[pl.ds(i * 16, 16)] = jnp.full((16,), 1.0, jnp.float32)

        @pl.when(sid == 0)
        def _():
            def zrow(i, c):
                z_v[pl.ds(i * 16, 16)] = jnp.zeros((16,), jnp.float32)
                return c
            lax.fori_loop(0, NPAD // 16, zrow, 0)
            pltpu.sync_copy(z_v, acc_sh)

        plsc.subcore_barrier()

        def body(j, c):
            pltpu.sync_copy(ones_v, acc_sh.at[dst_v.at[j]], add=True)
            return c
        lax.fori_loop(0, NCHUNK, body, 0)
        plsc.subcore_barrier()
        pltpu.sync_copy(acc_sh.at[pl.ds(sid * DSLAB, DSLAB)],
                        out_hbm.at[cid, pl.ds(sid * DSLAB, DSLAB)])

    pl.run_scoped(
        run,
        pltpu.VMEM((NCHUNK, CH), jnp.int32),
        pltpu.VMEM((CH,), jnp.float32),
        pltpu.VMEM((NPAD,), jnp.float32),
        pltpu.SemaphoreType.DMA,
    )


def _make_agg_kernel(d):
    @functools.partial(
        pl.kernel,
        out_type=jax.ShapeDtypeStruct((NC, N, d), jnp.float32),
        mesh=_mesh,
        compiler_params=pltpu.CompilerParams(use_tc_tiling_on_sc=False),
        scratch_types=[pltpu.VMEM_SHARED((N, d), jnp.float32)],
    )
    def agg(h_hbm, src_hbm, dst_hbm, out_hbm, acc_sh):
        cid = lax.axis_index("c")
        sid = lax.axis_index("s")
        wid = sid * NC + cid

        def run(src_v, dst_v, rows_v, sem):
            pltpu.async_copy(src_hbm.at[wid], src_v, sem).wait()
            pltpu.async_copy(dst_hbm.at[wid], dst_v, sem).wait()
            pltpu.sync_copy(h_hbm.at[pl.ds(sid * SLAB, SLAB)],
                            acc_sh.at[pl.ds(sid * SLAB, SLAB)])

            @pl.when(sid == 0)
            def _():
                pltpu.sync_copy(h_hbm.at[pl.ds(TAIL_OFF, TAIL)],
                                acc_sh.at[pl.ds(TAIL_OFF, TAIL)])

            plsc.subcore_barrier()

            def body(j, c):
                pltpu.async_copy(h_hbm.at[src_v.at[j]], rows_v, sem).wait()
                pltpu.sync_copy(rows_v, acc_sh.at[dst_v.at[j]], add=True)
                return c
            lax.fori_loop(0, NCHUNK, body, 0)
            plsc.subcore_barrier()
            pltpu.sync_copy(acc_sh.at[pl.ds(sid * SLAB, SLAB)],
                            out_hbm.at[cid, pl.ds(sid * SLAB, SLAB)])

            @pl.when(sid == 0)
            def _():
                pltpu.sync_copy(acc_sh.at[pl.ds(TAIL_OFF, TAIL)],
                                out_hbm.at[cid, pl.ds(TAIL_OFF, TAIL)])

        pl.run_scoped(
            run,
            pltpu.VMEM((NCHUNK, CH), jnp.int32),
            pltpu.VMEM((NCHUNK, CH), jnp.int32),
            pltpu.VMEM((CH, d), jnp.float32),
            pltpu.SemaphoreType.DMA,
        )

    return agg


_agg128 = _make_agg_kernel(HID)
_agg16 = _make_agg_kernel(NCLS)


_R = 1000


def _tc1_body(deg2_ref, x_ref, w1_ref, h1s_ref, dinv_ref):
    deg = deg2_ref[0] + deg2_ref[1] + 1.0
    dinv = lax.rsqrt(deg)
    dinv_ref[...] = dinv
    h = jnp.dot(x_ref[...], w1_ref[...], preferred_element_type=jnp.float32)
    h1s_ref[...] = h * dinv


def _tc1(deg2, x, W1):
    return pl.pallas_call(
        _tc1_body,
        grid=(N // _R,),
        in_specs=[
            pl.BlockSpec((NC, _R, 1), lambda i: (0, i, 0)),
            pl.BlockSpec((_R, D_IN), lambda i: (i, 0)),
            pl.BlockSpec((D_IN, HID), lambda i: (0, 0)),
        ],
        out_specs=[
            pl.BlockSpec((_R, HID), lambda i: (i, 0)),
            pl.BlockSpec((_R, 1), lambda i: (i, 0)),
        ],
        out_shape=[
            jax.ShapeDtypeStruct((N, HID), jnp.float32),
            jax.ShapeDtypeStruct((N, 1), jnp.float32),
        ],
    )(deg2, x, W1)


def _tc2_body(p_ref, h1s_ref, dinv_ref, b1_ref, w2_ref, h2s_ref):
    agg = p_ref[0] + p_ref[1] - h1s_ref[...]
    h1 = jnp.maximum(agg * dinv_ref[...] + b1_ref[...], 0.0)
    h2 = jnp.dot(h1, w2_ref[...], preferred_element_type=jnp.float32)
    h2s_ref[...] = h2 * dinv_ref[...]


def _tc2(p1, h1s, dinv, b1, W2):
    return pl.pallas_call(
        _tc2_body,
        grid=(N // _R,),
        in_specs=[
            pl.BlockSpec((NC, _R, HID), lambda i: (0, i, 0)),
            pl.BlockSpec((_R, HID), lambda i: (i, 0)),
            pl.BlockSpec((_R, 1), lambda i: (i, 0)),
            pl.BlockSpec((1, HID), lambda i: (0, 0)),
            pl.BlockSpec((HID, NCLS), lambda i: (0, 0)),
        ],
        out_specs=pl.BlockSpec((_R, NCLS), lambda i: (i, 0)),
        out_shape=jax.ShapeDtypeStruct((N, NCLS), jnp.float32),
    )(p1, h1s, dinv, b1, W2)


def _tc3_body(p_ref, h2s_ref, dinv_ref, b2_ref, out_ref):
    agg = p_ref[0] + p_ref[1] - h2s_ref[...]
    out_ref[...] = agg * dinv_ref[...] + b2_ref[...]


def _tc3(p2, h2s, dinv, b2):
    return pl.pallas_call(
        _tc3_body,
        grid=(N // _R,),
        in_specs=[
            pl.BlockSpec((NC, _R, NCLS), lambda i: (0, i, 0)),
            pl.BlockSpec((_R, NCLS), lambda i: (i, 0)),
            pl.BlockSpec((_R, 1), lambda i: (i, 0)),
            pl.BlockSpec((1, NCLS), lambda i: (0, 0)),
        ],
        out_specs=pl.BlockSpec((_R, NCLS), lambda i: (i, 0)),
        out_shape=jax.ShapeDtypeStruct((N, NCLS), jnp.float32),
    )(p2, h2s, dinv, b2)


def kernel(x, edge_index, W1, b1, W2, b2):
    src = edge_index[0].astype(jnp.int32).reshape(NW, NCHUNK, CH)
    dst = edge_index[1].astype(jnp.int32).reshape(NW, NCHUNK, CH)

    deg2 = _deg_kernel(dst)
    deg2 = deg2.reshape(NC, NPAD, 1)

    h1s, dinv = _tc1(deg2, x, W1)
    p1 = _agg128(h1s, src, dst)
    h2s = _tc2(p1, h1s, dinv, b1.reshape(1, HID), W2)
    p2 = _agg16(h2s, src, dst)
    out = _tc3(p2, h2s, dinv, b2.reshape(1, NCLS))
    return out


# speedup vs baseline: 37.5500x; 1.6328x over previous
import functools

import jax
import jax.numpy as jnp
from jax import lax
from jax.experimental import pallas as pl
from jax.experimental.pallas import tpu as pltpu
from jax.experimental.pallas import tpu_sc as plsc

N = 10000
E = 320000
D_IN = 128
HID = 128
NCLS = 16

NC = 2
NS = 16
NW = NC * NS
EPW = E // NW
CH = 32
PADE = 240
EPWP = EPW + PADE
NCHUNK = EPWP // CH
NDUM = 64
NACC = N + NDUM
SLAB = 624
TAIL = N - NS * SLAB
TAIL_OFF = NS * SLAB
NPAD = 10240
DSLAB = NPAD // NS
NBUF = 5
LA = 4

_mesh = plsc.VectorSubcoreMesh(core_axis_name="c", subcore_axis_name="s")


@functools.partial(
    pl.kernel,
    out_type=jax.ShapeDtypeStruct((NC, NPAD), jnp.float32),
    mesh=_mesh,
    compiler_params=pltpu.CompilerParams(use_tc_tiling_on_sc=False),
    scratch_types=[pltpu.VMEM_SHARED((NPAD,), jnp.float32)],
)
def _deg_kernel(dst_hbm, out_hbm, acc_sh):
    cid = lax.axis_index("c")
    sid = lax.axis_index("s")
    wid = sid * NC + cid

    def run(dst_v, ones_v, z_v, sem):
        pltpu.async_copy(dst_hbm.at[wid], dst_v, sem).wait()
        for i in range(CH // 16):
            ones_v[pl.ds(i * 16, 16)] = jnp.full((16,), 1.0, jnp.float32)

        @pl.when(sid == 0)
        def _():
            def zrow(i, c):
                z_v[pl.ds(i * 16, 16)] = jnp.zeros((16,), jnp.float32)
                return c
            lax.fori_loop(0, NPAD // 16, zrow, 0)
            pltpu.sync_copy(z_v, acc_sh)

        plsc.subcore_barrier()

        grp = 32
        def group(gi, c):
            def fire(j, c2):
                pltpu.async_copy(ones_v, acc_sh.at[dst_v.at[gi * grp + j]],
                                 sem, add=True)
                return c2
            lax.fori_loop(0, grp, fire, 0)

            def drain(j, c2):
                pltpu.make_async_copy(ones_v, acc_sh.at[dst_v.at[gi * grp + j]],
                                      sem).wait()
                return c2
            lax.fori_loop(0, grp, drain, 0)
            return c
        lax.fori_loop(0, NCHUNK // grp, group, 0)
        plsc.subcore_barrier()
        pltpu.sync_copy(acc_sh.at[pl.ds(sid * DSLAB, DSLAB)],
                        out_hbm.at[cid, pl.ds(sid * DSLAB, DSLAB)])

    pl.run_scoped(
        run,
        pltpu.VMEM((NCHUNK, CH), jnp.int32),
        pltpu.VMEM((CH,), jnp.float32),
        pltpu.VMEM((NPAD,), jnp.float32),
        pltpu.SemaphoreType.DMA,
    )


def _make_agg_kernel(d):
    @functools.partial(
        pl.kernel,
        out_type=jax.ShapeDtypeStruct((NC, N, d), jnp.float32),
        mesh=_mesh,
        compiler_params=pltpu.CompilerParams(use_tc_tiling_on_sc=False),
        scratch_types=[pltpu.VMEM_SHARED((NACC, d), jnp.float32)],
    )
    def agg(h_hbm, src_hbm, dst_hbm, out_hbm, acc_sh):
        cid = lax.axis_index("c")
        sid = lax.axis_index("s")
        wid = sid * NC + cid

        def run(src_v, dst_v, sem,
                rows0, rows1, rows2, rows3, rows4,
                gs0, gs1, gs2, gs3, gs4,
                ss0, ss1, ss2, ss3, ss4):
            rows = [rows0, rows1, rows2, rows3, rows4]
            gs = [gs0, gs1, gs2, gs3, gs4]
            ss = [ss0, ss1, ss2, ss3, ss4]
            pltpu.async_copy(src_hbm.at[wid], src_v, sem).wait()
            pltpu.async_copy(dst_hbm.at[wid], dst_v, sem).wait()
            for b in range(LA):
                pltpu.async_copy(h_hbm.at[src_v.at[b]], rows[b], gs[b])
            pltpu.sync_copy(h_hbm.at[pl.ds(sid * SLAB, SLAB)],
                            acc_sh.at[pl.ds(sid * SLAB, SLAB)])

            @pl.when(sid == 0)
            def _():
                pltpu.sync_copy(h_hbm.at[pl.ds(TAIL_OFF, TAIL)],
                                acc_sh.at[pl.ds(TAIL_OFF, TAIL)])

            plsc.subcore_barrier()

            def outer(i, c):
                g0 = i * NBUF
                for b in range(NBUF):
                    g = g0 + b
                    bb = (b + LA) % NBUF
                    gla = g + LA

                    @pl.when(gla < NCHUNK)
                    def _():
                        @pl.when(gla >= NBUF)
                        def _():
                            pltpu.make_async_copy(
                                rows[bb], acc_sh.at[dst_v.at[gla - NBUF]],
                                ss[bb]).wait()
                        pltpu.async_copy(h_hbm.at[src_v.at[gla]],
                                         rows[bb], gs[bb])

                    pltpu.make_async_copy(h_hbm.at[src_v.at[g]],
                                          rows[b], gs[b]).wait()
                    pltpu.async_copy(rows[b], acc_sh.at[dst_v.at[g]],
                                     ss[b], add=True)
                return c
            lax.fori_loop(0, NCHUNK // NBUF, outer, 0)
            for b in range(NBUF):
                g = NCHUNK - NBUF + b
                pltpu.make_async_copy(rows[b], acc_sh.at[dst_v.at[g]],
                                      ss[b]).wait()
            plsc.subcore_barrier()
            pltpu.sync_copy(acc_sh.at[pl.ds(sid * SLAB, SLAB)],
                            out_hbm.at[cid, pl.ds(sid * SLAB, SLAB)])

            @pl.when(sid == 0)
            def _():
                pltpu.sync_copy(acc_sh.at[pl.ds(TAIL_OFF, TAIL)],
                                out_hbm.at[cid, pl.ds(TAIL_OFF, TAIL)])

        pl.run_scoped(
            run,
            pltpu.VMEM((NCHUNK, CH), jnp.int32),
            pltpu.VMEM((NCHUNK, CH), jnp.int32),
            pltpu.SemaphoreType.DMA,
            *[pltpu.VMEM((CH, d), jnp.float32) for _ in range(NBUF)],
            *[pltpu.SemaphoreType.DMA for _ in range(2 * NBUF)],
        )

    return agg


_agg128 = _make_agg_kernel(HID)
_agg16 = _make_agg_kernel(NCLS)


_R = 1000


def _tc1_body(deg2_ref, x_ref, w1_ref, h1s_ref, dinv_ref):
    deg = deg2_ref[0] + deg2_ref[1] + 1.0
    dinv = lax.rsqrt(deg)
    dinv_ref[...] = dinv
    h = jnp.dot(x_ref[...], w1_ref[...], preferred_element_type=jnp.float32)
    h1s_ref[...] = h * dinv


def _tc1(deg2, x, W1):
    return pl.pallas_call(
        _tc1_body,
        grid=(N // _R,),
        in_specs=[
            pl.BlockSpec((NC, _R, 1), lambda i: (0, i, 0)),
            pl.BlockSpec((_R, D_IN), lambda i: (i, 0)),
            pl.BlockSpec((D_IN, HID), lambda i: (0, 0)),
        ],
        out_specs=[
            pl.BlockSpec((_R, HID), lambda i: (i, 0)),
            pl.BlockSpec((_R, 1), lambda i: (i, 0)),
        ],
        out_shape=[
            jax.ShapeDtypeStruct((N, HID), jnp.float32),
            jax.ShapeDtypeStruct((N, 1), jnp.float32),
        ],
    )(deg2, x, W1)


def _tc2_body(p_ref, h1s_ref, dinv_ref, b1_ref, w2_ref, h2s_ref):
    agg = p_ref[0] + p_ref[1] - h1s_ref[...]
    h1 = jnp.maximum(agg * dinv_ref[...] + b1_ref[...], 0.0)
    h2 = jnp.dot(h1, w2_ref[...], preferred_element_type=jnp.float32)
    h2s_ref[...] = h2 * dinv_ref[...]


def _tc2(p1, h1s, dinv, b1, W2):
    return pl.pallas_call(
        _tc2_body,
        grid=(N // _R,),
        in_specs=[
            pl.BlockSpec((NC, _R, HID), lambda i: (0, i, 0)),
            pl.BlockSpec((_R, HID), lambda i: (i, 0)),
            pl.BlockSpec((_R, 1), lambda i: (i, 0)),
            pl.BlockSpec((1, HID), lambda i: (0, 0)),
            pl.BlockSpec((HID, NCLS), lambda i: (0, 0)),
        ],
        out_specs=pl.BlockSpec((_R, NCLS), lambda i: (i, 0)),
        out_shape=jax.ShapeDtypeStruct((N, NCLS), jnp.float32),
    )(p1, h1s, dinv, b1, W2)


def _tc3_body(p_ref, h2s_ref, dinv_ref, b2_ref, out_ref):
    agg = p_ref[0] + p_ref[1] - h2s_ref[...]
    out_ref[...] = agg * dinv_ref[...] + b2_ref[...]


def _tc3(p2, h2s, dinv, b2):
    return pl.pallas_call(
        _tc3_body,
        grid=(N // _R,),
        in_specs=[
            pl.BlockSpec((NC, _R, NCLS), lambda i: (0, i, 0)),
            pl.BlockSpec((_R, NCLS), lambda i: (i, 0)),
            pl.BlockSpec((_R, 1), lambda i: (i, 0)),
            pl.BlockSpec((1, NCLS), lambda i: (0, 0)),
        ],
        out_specs=pl.BlockSpec((_R, NCLS), lambda i: (i, 0)),
        out_shape=jax.ShapeDtypeStruct((N, NCLS), jnp.float32),
    )(p2, h2s, dinv, b2)


def kernel(x, edge_index, W1, b1, W2, b2):
    srcw = edge_index[0].astype(jnp.int32).reshape(NW, EPW)
    dstw = edge_index[1].astype(jnp.int32).reshape(NW, EPW)
    padi = jnp.arange(PADE, dtype=jnp.int32)
    spad = jnp.broadcast_to((padi * 131) % N, (NW, PADE))
    dpad = jnp.broadcast_to(N + (padi % NDUM), (NW, PADE))
    src = jnp.concatenate([srcw, spad], axis=1).reshape(NW, NCHUNK, CH)
    dst = jnp.concatenate([dstw, dpad], axis=1).reshape(NW, NCHUNK, CH)

    deg2 = _deg_kernel(dst)
    deg2 = deg2.reshape(NC, NPAD, 1)

    h1s, dinv = _tc1(deg2, x, W1)
    p1 = _agg128(h1s, src, dst)
    h2s = _tc2(p1, h1s, dinv, b1.reshape(1, HID), W2)
    p2 = _agg16(h2s, src, dst)
    out = _tc3(p2, h2s, dinv, b2.reshape(1, NCLS))
    return out


# speedup vs baseline: 42.2881x; 1.1262x over previous
import functools

import jax
import jax.numpy as jnp
from jax import lax
from jax.experimental import pallas as pl
from jax.experimental.pallas import tpu as pltpu
from jax.experimental.pallas import tpu_sc as plsc

N = 10000
E = 320000
D_IN = 128
HID = 128
NCLS = 16

NC = 2
NS = 16
NW = NC * NS
EPW = E // NW
CH = 32
PADE = 240
EPWP = EPW + PADE
NCHUNK = EPWP // CH
NDUM = 64
NACC = N + NDUM
SLAB = 624
TAIL = N - NS * SLAB
TAIL_OFF = NS * SLAB
NPAD = 10240
DSLAB = NPAD // NS
NBUF = 5
LA = 4

_mesh = plsc.VectorSubcoreMesh(core_axis_name="c", subcore_axis_name="s")


@functools.partial(
    pl.kernel,
    out_type=jax.ShapeDtypeStruct((NC, NPAD), jnp.float32),
    mesh=_mesh,
    compiler_params=pltpu.CompilerParams(use_tc_tiling_on_sc=False),
    scratch_types=[pltpu.VMEM_SHARED((NPAD,), jnp.float32)],
)
def _deg_kernel(dst_hbm, out_hbm, acc_sh):
    cid = lax.axis_index("c")
    sid = lax.axis_index("s")
    wid = sid * NC + cid

    def run(dst_v, ones_v, z_v, sem):
        pltpu.async_copy(dst_hbm.at[wid], dst_v, sem).wait()
        for i in range(CH // 16):
            ones_v[pl.ds(i * 16, 16)] = jnp.full((16,), 1.0, jnp.float32)

        @pl.when(sid == 0)
        def _():
            def zrow(i, c):
                z_v[pl.ds(i * 16, 16)] = jnp.zeros((16,), jnp.float32)
                return c
            lax.fori_loop(0, NPAD // 16, zrow, 0)
            pltpu.sync_copy(z_v, acc_sh)

        plsc.subcore_barrier()

        grp = 32
        def group(gi, c):
            def fire(j, c2):
                pltpu.async_copy(ones_v, acc_sh.at[dst_v.at[gi * grp + j]],
                                 sem, add=True)
                return c2
            lax.fori_loop(0, grp, fire, 0)

            def drain(j, c2):
                pltpu.make_async_copy(ones_v, acc_sh.at[dst_v.at[gi * grp + j]],
                                      sem).wait()
                return c2
            lax.fori_loop(0, grp, drain, 0)
            return c
        lax.fori_loop(0, NCHUNK // grp, group, 0)
        plsc.subcore_barrier()
        pltpu.sync_copy(acc_sh.at[pl.ds(sid * DSLAB, DSLAB)],
                        out_hbm.at[cid, pl.ds(sid * DSLAB, DSLAB)])

    pl.run_scoped(
        run,
        pltpu.VMEM((NCHUNK, CH), jnp.int32),
        pltpu.VMEM((CH,), jnp.float32),
        pltpu.VMEM((NPAD,), jnp.float32),
        pltpu.SemaphoreType.DMA,
    )


def _make_agg_kernel(d, ch):
    nchunk = EPWP // ch
    @functools.partial(
        pl.kernel,
        out_type=jax.ShapeDtypeStruct((NC, N, d), jnp.float32),
        mesh=_mesh,
        compiler_params=pltpu.CompilerParams(use_tc_tiling_on_sc=False),
        scratch_types=[pltpu.VMEM_SHARED((NACC, d), jnp.float32)],
    )
    def agg(h_hbm, src_hbm, dst_hbm, out_hbm, acc_sh):
        cid = lax.axis_index("c")
        sid = lax.axis_index("s")
        wid = sid * NC + cid

        def run(src_v, dst_v, sem,
                rows0, rows1, rows2, rows3, rows4,
                gs0, gs1, gs2, gs3, gs4,
                ss0, ss1, ss2, ss3, ss4):
            rows = [rows0, rows1, rows2, rows3, rows4]
            gs = [gs0, gs1, gs2, gs3, gs4]
            ss = [ss0, ss1, ss2, ss3, ss4]
            pltpu.async_copy(src_hbm.at[wid], src_v, sem).wait()
            pltpu.async_copy(dst_hbm.at[wid], dst_v, sem).wait()
            for b in range(LA):
                pltpu.async_copy(h_hbm.at[src_v.at[b]], rows[b], gs[b])
            pltpu.sync_copy(h_hbm.at[pl.ds(sid * SLAB, SLAB)],
                            acc_sh.at[pl.ds(sid * SLAB, SLAB)])

            @pl.when(sid == 0)
            def _():
                pltpu.sync_copy(h_hbm.at[pl.ds(TAIL_OFF, TAIL)],
                                acc_sh.at[pl.ds(TAIL_OFF, TAIL)])

            plsc.subcore_barrier()

            def outer(i, c):
                g0 = i * NBUF
                for b in range(NBUF):
                    g = g0 + b
                    bb = (b + LA) % NBUF
                    gla = g + LA

                    @pl.when(gla < nchunk)
                    def _():
                        @pl.when(gla >= NBUF)
                        def _():
                            pltpu.make_async_copy(
                                rows[bb], acc_sh.at[dst_v.at[gla - NBUF]],
                                ss[bb]).wait()
                        pltpu.async_copy(h_hbm.at[src_v.at[gla]],
                                         rows[bb], gs[bb])

                    pltpu.make_async_copy(h_hbm.at[src_v.at[g]],
                                          rows[b], gs[b]).wait()
                    pltpu.async_copy(rows[b], acc_sh.at[dst_v.at[g]],
                                     ss[b], add=True)
                return c
            lax.fori_loop(0, nchunk // NBUF, outer, 0)
            for b in range(NBUF):
                g = nchunk - NBUF + b
                pltpu.make_async_copy(rows[b], acc_sh.at[dst_v.at[g]],
                                      ss[b]).wait()
            plsc.subcore_barrier()
            pltpu.sync_copy(acc_sh.at[pl.ds(sid * SLAB, SLAB)],
                            out_hbm.at[cid, pl.ds(sid * SLAB, SLAB)])

            @pl.when(sid == 0)
            def _():
                pltpu.sync_copy(acc_sh.at[pl.ds(TAIL_OFF, TAIL)],
                                out_hbm.at[cid, pl.ds(TAIL_OFF, TAIL)])

        pl.run_scoped(
            run,
            pltpu.VMEM((nchunk, ch), jnp.int32),
            pltpu.VMEM((nchunk, ch), jnp.int32),
            pltpu.SemaphoreType.DMA,
            *[pltpu.VMEM((ch, d), jnp.float32) for _ in range(NBUF)],
            *[pltpu.SemaphoreType.DMA for _ in range(2 * NBUF)],
        )

    return agg


_agg128 = _make_agg_kernel(HID, CH)
_agg16 = _make_agg_kernel(NCLS, 128)


_R = 1000


def _tc1_body(deg2_ref, x_ref, w1_ref, h1s_ref, dinv_ref):
    deg = deg2_ref[0] + deg2_ref[1] + 1.0
    dinv = lax.rsqrt(deg)
    dinv_ref[...] = dinv
    h = jnp.dot(x_ref[...], w1_ref[...], preferred_element_type=jnp.float32)
    h1s_ref[...] = h * dinv


def _tc1(deg2, x, W1):
    return pl.pallas_call(
        _tc1_body,
        grid=(N // _R,),
        in_specs=[
            pl.BlockSpec((NC, _R, 1), lambda i: (0, i, 0)),
            pl.BlockSpec((_R, D_IN), lambda i: (i, 0)),
            pl.BlockSpec((D_IN, HID), lambda i: (0, 0)),
        ],
        out_specs=[
            pl.BlockSpec((_R, HID), lambda i: (i, 0)),
            pl.BlockSpec((_R, 1), lambda i: (i, 0)),
        ],
        out_shape=[
            jax.ShapeDtypeStruct((N, HID), jnp.float32),
            jax.ShapeDtypeStruct((N, 1), jnp.float32),
        ],
    )(deg2, x, W1)


def _tc2_body(p_ref, h1s_ref, dinv_ref, b1_ref, w2_ref, h2s_ref):
    agg = p_ref[0] + p_ref[1] - h1s_ref[...]
    h1 = jnp.maximum(agg * dinv_ref[...] + b1_ref[...], 0.0)
    h2 = jnp.dot(h1, w2_ref[...], preferred_element_type=jnp.float32)
    h2s_ref[...] = h2 * dinv_ref[...]


def _tc2(p1, h1s, dinv, b1, W2):
    return pl.pallas_call(
        _tc2_body,
        grid=(N // _R,),
        in_specs=[
            pl.BlockSpec((NC, _R, HID), lambda i: (0, i, 0)),
            pl.BlockSpec((_R, HID), lambda i: (i, 0)),
            pl.BlockSpec((_R, 1), lambda i: (i, 0)),
            pl.BlockSpec((1, HID), lambda i: (0, 0)),
            pl.BlockSpec((HID, NCLS), lambda i: (0, 0)),
        ],
        out_specs=pl.BlockSpec((_R, NCLS), lambda i: (i, 0)),
        out_shape=jax.ShapeDtypeStruct((N, NCLS), jnp.float32),
    )(p1, h1s, dinv, b1, W2)


def _tc3_body(p_ref, h2s_ref, dinv_ref, b2_ref, out_ref):
    agg = p_ref[0] + p_ref[1] - h2s_ref[...]
    out_ref[...] = agg * dinv_ref[...] + b2_ref[...]


def _tc3(p2, h2s, dinv, b2):
    return pl.pallas_call(
        _tc3_body,
        grid=(N // _R,),
        in_specs=[
            pl.BlockSpec((NC, _R, NCLS), lambda i: (0, i, 0)),
            pl.BlockSpec((_R, NCLS), lambda i: (i, 0)),
            pl.BlockSpec((_R, 1), lambda i: (i, 0)),
            pl.BlockSpec((1, NCLS), lambda i: (0, 0)),
        ],
        out_specs=pl.BlockSpec((_R, NCLS), lambda i: (i, 0)),
        out_shape=jax.ShapeDtypeStruct((N, NCLS), jnp.float32),
    )(p2, h2s, dinv, b2)


def kernel(x, edge_index, W1, b1, W2, b2):
    srcw = edge_index[0].astype(jnp.int32).reshape(NW, EPW)
    dstw = edge_index[1].astype(jnp.int32).reshape(NW, EPW)
    padi = jnp.arange(PADE, dtype=jnp.int32)
    spad = jnp.broadcast_to((padi * 131) % N, (NW, PADE))
    dpad = jnp.broadcast_to(N + (padi % NDUM), (NW, PADE))
    srcp = jnp.concatenate([srcw, spad], axis=1)
    dstp = jnp.concatenate([dstw, dpad], axis=1)
    src = srcp.reshape(NW, NCHUNK, CH)
    dst = dstp.reshape(NW, NCHUNK, CH)
    srcL = srcp.reshape(NW, EPWP // 128, 128)
    dstL = dstp.reshape(NW, EPWP // 128, 128)

    deg2 = _deg_kernel(dst)
    deg2 = deg2.reshape(NC, NPAD, 1)

    h1s, dinv = _tc1(deg2, x, W1)
    p1 = _agg128(h1s, src, dst)
    h2s = _tc2(p1, h1s, dinv, b1.reshape(1, HID), W2)
    p2 = _agg16(h2s, srcL, dstL)
    out = _tc3(p2, h2s, dinv, b2.reshape(1, NCLS))
    return out


# speedup vs baseline: 47.8544x; 1.1316x over previous
import functools

import jax
import jax.numpy as jnp
from jax import lax
from jax.experimental import pallas as pl
from jax.experimental.pallas import tpu as pltpu
from jax.experimental.pallas import tpu_sc as plsc

N = 10000
E = 320000
D_IN = 128
HID = 128
NCLS = 16

NC = 2
NS = 16
NW = NC * NS
EPW = E // NW
CH = 32
PADE = 240
EPWP = EPW + PADE
NCHUNK = EPWP // CH
NDUM = 64
NACC = N + NDUM
SLAB = 624
TAIL = N - NS * SLAB
TAIL_OFF = NS * SLAB
NPAD = 10240
DSLAB = NPAD // NS
NBUF = 5
LA = 4

_mesh = plsc.VectorSubcoreMesh(core_axis_name="c", subcore_axis_name="s")


@functools.partial(
    pl.kernel,
    out_type=jax.ShapeDtypeStruct((NC, NPAD), jnp.float32),
    mesh=_mesh,
    compiler_params=pltpu.CompilerParams(use_tc_tiling_on_sc=False),
    scratch_types=[pltpu.VMEM_SHARED((NPAD,), jnp.float32)],
)
def _deg_kernel(dst_hbm, out_hbm, acc_sh):
    cid = lax.axis_index("c")
    sid = lax.axis_index("s")
    wid = sid * NC + cid

    def run(dst_v, ones_v, z_v, sem):
        pltpu.async_copy(dst_hbm.at[wid], dst_v, sem).wait()
        for i in range(CH // 16):
            ones_v[pl.ds(i * 16, 16)] = jnp.full((16,), 1.0, jnp.float32)

        @pl.when(sid == 0)
        def _():
            def zrow(i, c):
                z_v[pl.ds(i * 16, 16)] = jnp.zeros((16,), jnp.float32)
                return c
            lax.fori_loop(0, NPAD // 16, zrow, 0)
            pltpu.sync_copy(z_v, acc_sh)

        plsc.subcore_barrier()

        grp = 32
        def group(gi, c):
            def fire(j, c2):
                pltpu.async_copy(ones_v, acc_sh.at[dst_v.at[gi * grp + j]],
                                 sem, add=True)
                return c2
            lax.fori_loop(0, grp, fire, 0)

            def drain(j, c2):
                pltpu.make_async_copy(ones_v, acc_sh.at[dst_v.at[gi * grp + j]],
                                      sem).wait()
                return c2
            lax.fori_loop(0, grp, drain, 0)
            return c
        lax.fori_loop(0, NCHUNK // grp, group, 0)
        plsc.subcore_barrier()
        pltpu.sync_copy(acc_sh.at[pl.ds(sid * DSLAB, DSLAB)],
                        out_hbm.at[cid, pl.ds(sid * DSLAB, DSLAB)])

    pl.run_scoped(
        run,
        pltpu.VMEM((NCHUNK, CH), jnp.int32),
        pltpu.VMEM((CH,), jnp.float32),
        pltpu.VMEM((NPAD,), jnp.float32),
        pltpu.SemaphoreType.DMA,
    )


def _make_agg_kernel(d, ch, nbuf, la):
    nchunk = EPWP // ch
    @functools.partial(
        pl.kernel,
        out_type=jax.ShapeDtypeStruct((NC, N, d), jnp.float32),
        mesh=_mesh,
        compiler_params=pltpu.CompilerParams(use_tc_tiling_on_sc=False),
        scratch_types=[pltpu.VMEM_SHARED((NACC, d), jnp.float32)],
    )
    def agg(h_hbm, src_hbm, dst_hbm, out_hbm, acc_sh):
        cid = lax.axis_index("c")
        sid = lax.axis_index("s")
        wid = sid * NC + cid

        def run(*refs):
            src_v, dst_v, sem = refs[0], refs[1], refs[2]
            rows = list(refs[3:3 + nbuf])
            gs = list(refs[3 + nbuf:3 + 2 * nbuf])
            ss = list(refs[3 + 2 * nbuf:3 + 3 * nbuf])
            pltpu.async_copy(src_hbm.at[wid], src_v, sem).wait()
            pltpu.async_copy(dst_hbm.at[wid], dst_v, sem).wait()
            for b in range(la):
                pltpu.async_copy(h_hbm.at[src_v.at[b]], rows[b], gs[b])
            pltpu.sync_copy(h_hbm.at[pl.ds(sid * SLAB, SLAB)],
                            acc_sh.at[pl.ds(sid * SLAB, SLAB)])

            @pl.when(sid == 0)
            def _():
                pltpu.sync_copy(h_hbm.at[pl.ds(TAIL_OFF, TAIL)],
                                acc_sh.at[pl.ds(TAIL_OFF, TAIL)])

            plsc.subcore_barrier()

            def outer(i, c):
                g0 = i * nbuf
                for b in range(nbuf):
                    g = g0 + b
                    bb = (b + la) % nbuf
                    gla = g + la

                    @pl.when(gla < nchunk)
                    def _():
                        @pl.when(gla >= nbuf)
                        def _():
                            pltpu.make_async_copy(
                                rows[bb], acc_sh.at[dst_v.at[gla - nbuf]],
                                ss[bb]).wait()
                        pltpu.async_copy(h_hbm.at[src_v.at[gla]],
                                         rows[bb], gs[bb])

                    pltpu.make_async_copy(h_hbm.at[src_v.at[g]],
                                          rows[b], gs[b]).wait()
                    pltpu.async_copy(rows[b], acc_sh.at[dst_v.at[g]],
                                     ss[b], add=True)
                return c
            lax.fori_loop(0, nchunk // nbuf, outer, 0)
            for b in range(nbuf):
                g = nchunk - nbuf + b
                pltpu.make_async_copy(rows[b], acc_sh.at[dst_v.at[g]],
                                      ss[b]).wait()
            plsc.subcore_barrier()
            pltpu.sync_copy(acc_sh.at[pl.ds(sid * SLAB, SLAB)],
                            out_hbm.at[cid, pl.ds(sid * SLAB, SLAB)])

            @pl.when(sid == 0)
            def _():
                pltpu.sync_copy(acc_sh.at[pl.ds(TAIL_OFF, TAIL)],
                                out_hbm.at[cid, pl.ds(TAIL_OFF, TAIL)])

        pl.run_scoped(
            run,
            pltpu.VMEM((nchunk, ch), jnp.int32),
            pltpu.VMEM((nchunk, ch), jnp.int32),
            pltpu.SemaphoreType.DMA,
            *[pltpu.VMEM((ch, d), jnp.float32) for _ in range(nbuf)],
            *[pltpu.SemaphoreType.DMA for _ in range(2 * nbuf)],
        )

    return agg


_agg128 = _make_agg_kernel(HID, CH, 5, 4)
_agg16 = _make_agg_kernel(NCLS, 128, 8, 7)


def _row_scale(dinv_row, ncols):
    ones = jnp.ones((1, ncols), jnp.float32)
    return lax.dot_general(dinv_row, ones, (((0,), (0,)), ((), ())),
                           preferred_element_type=jnp.float32)


def _tc1_body(deg2_ref, x_ref, w1_ref, h1s_ref, dinv_ref):
    deg = deg2_ref[0:1, :N] + deg2_ref[1:2, :N] + 1.0
    dinv = lax.rsqrt(deg)
    dinv_ref[...] = dinv
    h = jnp.dot(x_ref[...], w1_ref[...], preferred_element_type=jnp.float32)
    h1s_ref[...] = h * _row_scale(dinv, HID)


def _tc1(deg2, x, W1):
    return pl.pallas_call(
        _tc1_body,
        out_shape=[
            jax.ShapeDtypeStruct((N, HID), jnp.float32),
            jax.ShapeDtypeStruct((1, N), jnp.float32),
        ],
    )(deg2, x, W1)


def _tc2_body(p_ref, h1s_ref, dinv_ref, b1_ref, w2_ref, h2s_ref):
    dinv = dinv_ref[...]
    agg = p_ref[0] + p_ref[1] - h1s_ref[...]
    h1 = jnp.maximum(agg * _row_scale(dinv, HID) + b1_ref[...], 0.0)
    h2 = jnp.dot(h1, w2_ref[...], preferred_element_type=jnp.float32)
    h2s_ref[...] = h2 * _row_scale(dinv, NCLS)


def _tc2(p1, h1s, dinv, b1, W2):
    return pl.pallas_call(
        _tc2_body,
        out_shape=jax.ShapeDtypeStruct((N, NCLS), jnp.float32),
    )(p1, h1s, dinv, b1, W2)


def _tc3_body(p_ref, h2s_ref, dinv_ref, b2_ref, out_ref):
    agg = p_ref[0] + p_ref[1] - h2s_ref[...]
    out_ref[...] = agg * _row_scale(dinv_ref[...], NCLS) + b2_ref[...]


def _tc3(p2, h2s, dinv, b2):
    return pl.pallas_call(
        _tc3_body,
        out_shape=jax.ShapeDtypeStruct((N, NCLS), jnp.float32),
    )(p2, h2s, dinv, b2)


def kernel(x, edge_index, W1, b1, W2, b2):
    srcw = edge_index[0].astype(jnp.int32).reshape(NW, EPW)
    dstw = edge_index[1].astype(jnp.int32).reshape(NW, EPW)
    padi = jnp.arange(PADE, dtype=jnp.int32)
    spad = jnp.broadcast_to((padi * 131) % N, (NW, PADE))
    dpad = jnp.broadcast_to(N + (padi % NDUM), (NW, PADE))
    srcp = jnp.concatenate([srcw, spad], axis=1)
    dstp = jnp.concatenate([dstw, dpad], axis=1)
    src = srcp.reshape(NW, NCHUNK, CH)
    dst = dstp.reshape(NW, NCHUNK, CH)
    srcL = srcp.reshape(NW, EPWP // 128, 128)
    dstL = dstp.reshape(NW, EPWP // 128, 128)

    deg2 = _deg_kernel(dst)

    h1s, dinv = _tc1(deg2, x, W1)
    p1 = _agg128(h1s, src, dst)
    h2s = _tc2(p1, h1s, dinv, b1.reshape(1, HID), W2)
    p2 = _agg16(h2s, srcL, dstL)
    out = _tc3(p2, h2s, dinv, b2.reshape(1, NCLS))
    return out
